# Initial kernel scaffold; baseline (speedup 1.0000x reference)
import jax
import jax.numpy as jnp
from jax.experimental import pallas as pl


def kernel(edge_index, edge_weight, emb, W1_0, b1_0, W2_0, b2_0, W1_1, b1_1, W2_1, b2_1):
    raise NotImplementedError("write your pallas kernel here")


# speedup vs baseline: 2.5868x; 2.5868x over previous
import functools

import jax
import jax.numpy as jnp
from jax import lax
from jax.experimental import pallas as pl
from jax.experimental.pallas import tpu as pltpu
from jax.experimental.pallas import tpu_sc as plsc

N = 50000
E = 800000
D = 64

C = 80
EP = 819200
ROWSP = EP // C
NSC = 2
NTILE = 16
TROWS = ROWSP // NTILE
NB = 40
NBLK = TROWS // NB
OWN = N // NSC
HALF = 25088
WB = HALF // NTILE
ZR = 98


def _sc_scatter(src_r, dst_r, w_r, x):
    mesh = plsc.VectorSubcoreMesh(core_axis_name="c", subcore_axis_name="s",
                                  num_cores=NSC, num_subcores=NTILE)

    @functools.partial(
        pl.kernel,
        out_type=jax.ShapeDtypeStruct((NSC, HALF, D), jnp.float32),
        mesh=mesh,
        scratch_types=[
            pltpu.VMEM((NB, C), jnp.int32),
            pltpu.VMEM((NB, C), jnp.int32),
            pltpu.VMEM((NB, C), jnp.float32),
            pltpu.VMEM((C, D), jnp.float32),
            pltpu.VMEM((C,), jnp.int32),
            pltpu.VMEM((ZR, D), jnp.float32),
            pltpu.VMEM_SHARED((HALF, D), jnp.float32),
            pltpu.SemaphoreType.DMA,
            pltpu.SemaphoreType.DMA,
        ],
        compiler_params=pltpu.CompilerParams(use_tc_tiling_on_sc=False),
    )
    def k(src_hbm, dst_hbm, w_hbm, x_hbm, out_hbm,
          src_v, dst_v, w_v, rows_v, sidx_v, zbuf, accum, isem, gsem):
        cc = lax.axis_index("c")
        s = lax.axis_index("s")
        base = cc * OWN

        def zfill(r, carry):
            for g in range(D // 16):
                zbuf[r, pl.ds(g * 16, 16)] = jnp.zeros((16,), jnp.float32)
            return carry
        lax.fori_loop(0, ZR, zfill, 0)
        for z in range(WB // ZR):
            pltpu.sync_copy(zbuf, accum.at[pl.ds(s * WB + z * ZR, ZR)])
        plsc.subcore_barrier()

        def block_body(b, carry):
            r0 = s * TROWS + b * NB
            d1 = pltpu.async_copy(src_hbm.at[pl.ds(r0, NB)], src_v, isem)
            d2 = pltpu.async_copy(dst_hbm.at[pl.ds(r0, NB)], dst_v, isem)
            d3 = pltpu.async_copy(w_hbm.at[pl.ds(r0, NB)], w_v, isem)
            d1.wait()
            d2.wait()
            d3.wait()

            def chunk_body(j, inner):
                g = pltpu.async_copy(x_hbm.at[src_v.at[j]], rows_v, gsem)
                for v in range(C // 16):
                    dv = dst_v[j, pl.ds(v * 16, 16)]
                    loc = dv - base
                    ok = (loc >= 0) & (loc < OWN)
                    sidx_v[pl.ds(v * 16, 16)] = jnp.where(ok, loc, OWN)
                g.wait()
                for v16 in range(C // 16):
                    wv = w_v[j, pl.ds(v16 * 16, 16)]
                    for l in range(16):
                        e = v16 * 16 + l
                        ws = wv[l]
                        for v in range(D // 16):
                            sl = pl.ds(v * 16, 16)
                            rows_v[e, sl] = rows_v[e, sl] * ws
                pltpu.sync_copy(rows_v, accum.at[sidx_v], add=True)
                return inner
            lax.fori_loop(0, NB, chunk_body, 0)
            return carry
        lax.fori_loop(0, NBLK, block_body, 0)

        plsc.subcore_barrier()
        pltpu.sync_copy(accum.at[pl.ds(s * WB, WB)],
                        out_hbm.at[cc].at[pl.ds(s * WB, WB)])

    return k(src_r, dst_r, w_r, x)


BM = 1000


def _tc_dense(acc, x, W1, b1, W2, b2):
    blocks_per_half = OWN // BM

    def body(side_ref, x_ref, w1_ref, b1_ref, w2_ref, b2_ref, o_ref):
        sd = side_ref[0]
        xx = x_ref[...]
        y = jnp.dot(sd + xx, w1_ref[...], preferred_element_type=jnp.float32)
        y = y + jnp.dot(sd * xx, w2_ref[...], preferred_element_type=jnp.float32)
        y = y + b1_ref[...] + b2_ref[...]
        y = jnp.where(y >= 0, y, 0.01 * y)
        nrm = jnp.sqrt(jnp.sum(y * y, axis=1, keepdims=True))
        o_ref[...] = y / jnp.maximum(nrm, 1e-12)

    return pl.pallas_call(
        body,
        grid=(N // BM,),
        in_specs=[
            pl.BlockSpec((1, BM, D),
                         lambda i: (i // blocks_per_half, i % blocks_per_half, 0)),
            pl.BlockSpec((BM, D), lambda i: (i, 0)),
            pl.BlockSpec((D, D), lambda i: (0, 0)),
            pl.BlockSpec((1, D), lambda i: (0, 0)),
            pl.BlockSpec((D, D), lambda i: (0, 0)),
            pl.BlockSpec((1, D), lambda i: (0, 0)),
        ],
        out_specs=pl.BlockSpec((BM, D), lambda i: (i, 0)),
        out_shape=jax.ShapeDtypeStruct((N, D), jnp.float32),
    )(acc, x, W1, b1.reshape(1, D), W2, b2.reshape(1, D))


def kernel(edge_index, edge_weight, emb,
           W1_0, b1_0, W2_0, b2_0, W1_1, b1_1, W2_1, b2_1):
    pad = EP - E
    src_r = jnp.concatenate(
        [edge_index[0], jnp.zeros((pad,), jnp.int32)]).reshape(ROWSP, C)
    dst_r = jnp.concatenate(
        [edge_index[1], jnp.zeros((pad,), jnp.int32)]).reshape(ROWSP, C)
    w_r = jnp.concatenate(
        [edge_weight, jnp.zeros((pad,), jnp.float32)]).reshape(ROWSP, C)
    x = emb
    outs = [emb]
    for (W1, b1, W2, b2) in ((W1_0, b1_0, W2_0, b2_0),
                             (W1_1, b1_1, W2_1, b2_1)):
        acc = _sc_scatter(src_r, dst_r, w_r, x)
        x = _tc_dense(acc, x, W1, b1, W2, b2)
        outs.append(x)
    return jnp.concatenate(outs, axis=1)


# speedup vs baseline: 2.9309x; 1.1331x over previous
import functools

import jax
import jax.numpy as jnp
from jax import lax
from jax.experimental import pallas as pl
from jax.experimental.pallas import tpu as pltpu
from jax.experimental.pallas import tpu_sc as plsc

N = 50000
E = 800000
D = 64

C = 80
EP = 819200
ROWSP = EP // C
NSC = 2
NTILE = 16
TROWS = ROWSP // NTILE
NB = 40
NBLK = TROWS // NB
OWN = N // NSC
HALF = 25088
WB = HALF // NTILE
ZR = 98


def _sc_scatter(src_r, dst_r, w_r, x):
    mesh = plsc.VectorSubcoreMesh(core_axis_name="c", subcore_axis_name="s",
                                  num_cores=NSC, num_subcores=NTILE)

    @functools.partial(
        pl.kernel,
        out_type=jax.ShapeDtypeStruct((NSC, HALF, D), jnp.float32),
        mesh=mesh,
        scratch_types=[
            pltpu.VMEM((NB, C), jnp.int32),
            pltpu.VMEM((NB, C), jnp.int32),
            pltpu.VMEM((NB, C), jnp.float32),
            pltpu.VMEM((C, D), jnp.float32),
            pltpu.VMEM((C, D), jnp.float32),
            pltpu.VMEM((C,), jnp.int32),
            pltpu.VMEM((C,), jnp.int32),
            pltpu.VMEM((ZR, D), jnp.float32),
            pltpu.VMEM_SHARED((HALF, D), jnp.float32),
            pltpu.SemaphoreType.DMA,
            pltpu.SemaphoreType.DMA,
            pltpu.SemaphoreType.DMA,
        ],
        compiler_params=pltpu.CompilerParams(use_tc_tiling_on_sc=False),
    )
    def k(src_hbm, dst_hbm, w_hbm, x_hbm, out_hbm,
          src_v, dst_v, w_v, rows_a, rows_b, sidx_a, sidx_b, zbuf, accum,
          isem, gsem_a, gsem_b):
        cc = lax.axis_index("c")
        s = lax.axis_index("s")
        base = cc * OWN

        def zfill(r, carry):
            for g in range(D // 16):
                zbuf[r, pl.ds(g * 16, 16)] = jnp.zeros((16,), jnp.float32)
            return carry
        lax.fori_loop(0, ZR, zfill, 0)
        for z in range(WB // ZR):
            pltpu.sync_copy(zbuf, accum.at[pl.ds(s * WB + z * ZR, ZR)])
        plsc.subcore_barrier()

        def issue(j, rows_buf, gsem):
            pltpu.async_copy(x_hbm.at[src_v.at[j]], rows_buf, gsem)

        def process(j, rows_buf, sidx_buf, gsem):
            for v in range(C // 16):
                dv = dst_v[j, pl.ds(v * 16, 16)]
                loc = dv - base
                ok = (loc >= 0) & (loc < OWN)
                sidx_buf[pl.ds(v * 16, 16)] = jnp.where(ok, loc, OWN)
            pltpu.make_async_copy(x_hbm.at[src_v.at[j]], rows_buf, gsem).wait()
            for v16 in range(C // 16):
                wv = w_v[j, pl.ds(v16 * 16, 16)]
                for l in range(16):
                    e = v16 * 16 + l
                    ws = wv[l]
                    for v in range(D // 16):
                        sl = pl.ds(v * 16, 16)
                        rows_buf[e, sl] = rows_buf[e, sl] * ws
            pltpu.sync_copy(rows_buf, accum.at[sidx_buf], add=True)

        def block_body(b, carry):
            r0 = s * TROWS + b * NB
            d1 = pltpu.async_copy(src_hbm.at[pl.ds(r0, NB)], src_v, isem)
            d2 = pltpu.async_copy(dst_hbm.at[pl.ds(r0, NB)], dst_v, isem)
            d3 = pltpu.async_copy(w_hbm.at[pl.ds(r0, NB)], w_v, isem)
            d1.wait()
            d2.wait()
            d3.wait()

            issue(0, rows_a, gsem_a)

            def pair_body(p, inner):
                j0 = 2 * p
                issue(j0 + 1, rows_b, gsem_b)
                process(j0, rows_a, sidx_a, gsem_a)

                @pl.when(p < NB // 2 - 1)
                def _prefetch_a():
                    issue(j0 + 2, rows_a, gsem_a)
                process(j0 + 1, rows_b, sidx_b, gsem_b)
                return inner
            lax.fori_loop(0, NB // 2, pair_body, 0)
            return carry
        lax.fori_loop(0, NBLK, block_body, 0)

        plsc.subcore_barrier()
        pltpu.sync_copy(accum.at[pl.ds(s * WB, WB)],
                        out_hbm.at[cc].at[pl.ds(s * WB, WB)])

    return k(src_r, dst_r, w_r, x)


BM = 1000


def _tc_dense(acc, x, W1, b1, W2, b2):
    blocks_per_half = OWN // BM

    def body(side_ref, x_ref, w1_ref, b1_ref, w2_ref, b2_ref, o_ref):
        sd = side_ref[0]
        xx = x_ref[...]
        y = jnp.dot(sd + xx, w1_ref[...], preferred_element_type=jnp.float32)
        y = y + jnp.dot(sd * xx, w2_ref[...], preferred_element_type=jnp.float32)
        y = y + b1_ref[...] + b2_ref[...]
        y = jnp.where(y >= 0, y, 0.01 * y)
        nrm = jnp.sqrt(jnp.sum(y * y, axis=1, keepdims=True))
        o_ref[...] = y / jnp.maximum(nrm, 1e-12)

    return pl.pallas_call(
        body,
        grid=(N // BM,),
        in_specs=[
            pl.BlockSpec((1, BM, D),
                         lambda i: (i // blocks_per_half, i % blocks_per_half, 0)),
            pl.BlockSpec((BM, D), lambda i: (i, 0)),
            pl.BlockSpec((D, D), lambda i: (0, 0)),
            pl.BlockSpec((1, D), lambda i: (0, 0)),
            pl.BlockSpec((D, D), lambda i: (0, 0)),
            pl.BlockSpec((1, D), lambda i: (0, 0)),
        ],
        out_specs=pl.BlockSpec((BM, D), lambda i: (i, 0)),
        out_shape=jax.ShapeDtypeStruct((N, D), jnp.float32),
    )(acc, x, W1, b1.reshape(1, D), W2, b2.reshape(1, D))


def kernel(edge_index, edge_weight, emb,
           W1_0, b1_0, W2_0, b2_0, W1_1, b1_1, W2_1, b2_1):
    pad = EP - E
    src_r = jnp.concatenate(
        [edge_index[0], jnp.zeros((pad,), jnp.int32)]).reshape(ROWSP, C)
    dst_r = jnp.concatenate(
        [edge_index[1], jnp.zeros((pad,), jnp.int32)]).reshape(ROWSP, C)
    w_r = jnp.concatenate(
        [edge_weight, jnp.zeros((pad,), jnp.float32)]).reshape(ROWSP, C)
    x = emb
    outs = [emb]
    for (W1, b1, W2, b2) in ((W1_0, b1_0, W2_0, b2_0),
                             (W1_1, b1_1, W2_1, b2_1)):
        acc = _sc_scatter(src_r, dst_r, w_r, x)
        x = _tc_dense(acc, x, W1, b1, W2, b2)
        outs.append(x)
    return jnp.concatenate(outs, axis=1)


# speedup vs baseline: 3.0883x; 1.0537x over previous
import functools

import jax
import jax.numpy as jnp
from jax import lax
from jax.experimental import pallas as pl
from jax.experimental.pallas import tpu as pltpu
from jax.experimental.pallas import tpu_sc as plsc

N = 50000
E = 800000
D = 64

C = 80
EP = 819200
ROWSP = EP // C
NSC = 2
NTILE = 16
TROWS = ROWSP // NTILE
NB = 32
NBLK = TROWS // NB
OWN = N // NSC
HALF = 25088
WB = HALF // NTILE
ZR = 56
CB = NB * C + C


def _sc_scatter(src_r, dst_r, w_r, x):
    mesh = plsc.VectorSubcoreMesh(core_axis_name="c", subcore_axis_name="s",
                                  num_cores=NSC, num_subcores=NTILE)

    @functools.partial(
        pl.kernel,
        out_type=jax.ShapeDtypeStruct((NSC, HALF, D), jnp.float32),
        mesh=mesh,
        scratch_types=[
            pltpu.VMEM((NB, C), jnp.int32),
            pltpu.VMEM((NB, C), jnp.int32),
            pltpu.VMEM((NB, C), jnp.float32),
            pltpu.VMEM((CB,), jnp.int32),
            pltpu.VMEM((CB,), jnp.int32),
            pltpu.VMEM((CB,), jnp.float32),
            pltpu.VMEM((C, D), jnp.float32),
            pltpu.VMEM((C, D), jnp.float32),
            pltpu.VMEM((C,), jnp.int32),
            pltpu.VMEM((C,), jnp.int32),
            pltpu.VMEM((ZR, D), jnp.float32),
            pltpu.VMEM_SHARED((HALF, D), jnp.float32),
            pltpu.SemaphoreType.DMA,
            pltpu.SemaphoreType.DMA,
            pltpu.SemaphoreType.DMA,
        ],
        compiler_params=pltpu.CompilerParams(use_tc_tiling_on_sc=False,
                                             needs_layout_passes=False),
    )
    def k(src_hbm, dst_hbm, w_hbm, x_hbm, out_hbm,
          src_v, dst_v, w_v, src_c, dst_c, w_c, rows_a, rows_b,
          sidx_a, sidx_b, zbuf, accum, isem, gsem_a, gsem_b):
        cc = lax.axis_index("c")
        s = lax.axis_index("s")
        base = cc * OWN

        def zfill(r, carry):
            for g in range(D // 16):
                zbuf[r, pl.ds(g * 16, 16)] = jnp.zeros((16,), jnp.float32)
            return carry
        lax.fori_loop(0, ZR, zfill, 0)
        for z in range(WB // ZR):
            pltpu.sync_copy(zbuf, accum.at[pl.ds(s * WB + z * ZR, ZR)])
        plsc.subcore_barrier()

        def issue(j, rows_buf, gsem):
            pltpu.async_copy(x_hbm.at[src_c.at[pl.ds(j * C, C)]],
                             rows_buf, gsem)

        def process(j, rows_buf, sidx_buf, gsem):
            for v in range(C // 16):
                sl = pl.ds(v * 16, 16)
                sidx_buf[sl] = dst_c[pl.ds(j * C + v * 16, 16)]
            pltpu.make_async_copy(x_hbm.at[src_c.at[pl.ds(j * C, C)]],
                                  rows_buf, gsem).wait()
            for v16 in range(C // 16):
                wv = w_c[pl.ds(j * C + v16 * 16, 16)]
                for l in range(16):
                    e = v16 * 16 + l
                    ws = wv[l]
                    for v in range(D // 16):
                        sl = pl.ds(v * 16, 16)
                        rows_buf[e, sl] = rows_buf[e, sl] * ws
            pltpu.sync_copy(rows_buf, accum.at[sidx_buf], add=True)

        def block_body(b, carry):
            r0 = s * TROWS + b * NB
            d1 = pltpu.async_copy(src_hbm.at[pl.ds(r0, NB)], src_v, isem)
            d2 = pltpu.async_copy(dst_hbm.at[pl.ds(r0, NB)], dst_v, isem)
            d3 = pltpu.async_copy(w_hbm.at[pl.ds(r0, NB)], w_v, isem)
            d1.wait()
            d2.wait()
            d3.wait()

            def comp_row(r, off):
                for g in range(C // 16):
                    sl = pl.ds(g * 16, 16)
                    loc = dst_v[r, sl] - base
                    ok = (loc >= 0) & (loc < OWN)
                    plsc.store_compressed(src_c.at[pl.ds(off, 16)],
                                          src_v[r, sl], mask=ok)
                    plsc.store_compressed(dst_c.at[pl.ds(off, 16)],
                                          loc, mask=ok)
                    plsc.store_compressed(w_c.at[pl.ds(off, 16)],
                                          w_v[r, sl], mask=ok)
                    off = off + plsc.all_reduce_population_count(ok)[0]
                return off
            cnt = lax.fori_loop(0, NB, comp_row, jnp.int32(0))

            zi = jnp.zeros((16,), jnp.int32)
            zf = jnp.zeros((16,), jnp.float32)
            full = zi == zi
            for g in range(C // 16):
                plsc.store_compressed(src_c.at[pl.ds(cnt + g * 16, 16)],
                                      zi, mask=full)
                plsc.store_compressed(dst_c.at[pl.ds(cnt + g * 16, 16)],
                                      zi, mask=full)
                plsc.store_compressed(w_c.at[pl.ds(cnt + g * 16, 16)],
                                      zf, mask=full)

            nchunks = lax.div(cnt + (C - 1), jnp.int32(C))

            @pl.when(nchunks > 0)
            def _prime():
                issue(0, rows_a, gsem_a)

            def pair_body(p, inner):
                j0 = 2 * p
                issue(j0 + 1, rows_b, gsem_b)
                process(j0, rows_a, sidx_a, gsem_a)

                @pl.when(j0 + 2 < nchunks)
                def _prefetch_a():
                    issue(j0 + 2, rows_a, gsem_a)
                process(j0 + 1, rows_b, sidx_b, gsem_b)
                return inner
            lax.fori_loop(0, nchunks // 2, pair_body, 0)

            @pl.when(lax.rem(nchunks, jnp.int32(2)) == 1)
            def _last_odd():
                process(nchunks - 1, rows_a, sidx_a, gsem_a)
            return carry
        lax.fori_loop(0, NBLK, block_body, 0)

        plsc.subcore_barrier()
        pltpu.sync_copy(accum.at[pl.ds(s * WB, WB)],
                        out_hbm.at[cc].at[pl.ds(s * WB, WB)])

    return k(src_r, dst_r, w_r, x)


BM = 1000


def _tc_dense(acc, x, W1, b1, W2, b2):
    blocks_per_half = OWN // BM

    def body(side_ref, x_ref, w1_ref, b1_ref, w2_ref, b2_ref, o_ref):
        sd = side_ref[0]
        xx = x_ref[...]
        y = jnp.dot(sd + xx, w1_ref[...], preferred_element_type=jnp.float32)
        y = y + jnp.dot(sd * xx, w2_ref[...], preferred_element_type=jnp.float32)
        y = y + b1_ref[...] + b2_ref[...]
        y = jnp.where(y >= 0, y, 0.01 * y)
        nrm = jnp.sqrt(jnp.sum(y * y, axis=1, keepdims=True))
        o_ref[...] = y / jnp.maximum(nrm, 1e-12)

    return pl.pallas_call(
        body,
        grid=(N // BM,),
        in_specs=[
            pl.BlockSpec((1, BM, D),
                         lambda i: (i // blocks_per_half, i % blocks_per_half, 0)),
            pl.BlockSpec((BM, D), lambda i: (i, 0)),
            pl.BlockSpec((D, D), lambda i: (0, 0)),
            pl.BlockSpec((1, D), lambda i: (0, 0)),
            pl.BlockSpec((D, D), lambda i: (0, 0)),
            pl.BlockSpec((1, D), lambda i: (0, 0)),
        ],
        out_specs=pl.BlockSpec((BM, D), lambda i: (i, 0)),
        out_shape=jax.ShapeDtypeStruct((N, D), jnp.float32),
    )(acc, x, W1, b1.reshape(1, D), W2, b2.reshape(1, D))


def kernel(edge_index, edge_weight, emb,
           W1_0, b1_0, W2_0, b2_0, W1_1, b1_1, W2_1, b2_1):
    pad = EP - E
    src_r = jnp.concatenate(
        [edge_index[0], jnp.zeros((pad,), jnp.int32)]).reshape(ROWSP, C)
    dst_r = jnp.concatenate(
        [edge_index[1], jnp.zeros((pad,), jnp.int32)]).reshape(ROWSP, C)
    w_r = jnp.concatenate(
        [edge_weight, jnp.zeros((pad,), jnp.float32)]).reshape(ROWSP, C)
    x = emb
    outs = [emb]
    for (W1, b1, W2, b2) in ((W1_0, b1_0, W2_0, b2_0),
                             (W1_1, b1_1, W2_1, b2_1)):
        acc = _sc_scatter(src_r, dst_r, w_r, x)
        x = _tc_dense(acc, x, W1, b1, W2, b2)
        outs.append(x)
    return jnp.concatenate(outs, axis=1)


# speedup vs baseline: 3.3455x; 1.0833x over previous
import functools

import jax
import jax.numpy as jnp
from jax import lax
from jax.experimental import pallas as pl
from jax.experimental.pallas import tpu as pltpu
from jax.experimental.pallas import tpu_sc as plsc

N = 50000
E = 800000
D = 64

C = 80
NSC = 2
NTILE = 16
TE = E // (NTILE)
BLK = 1280
NBLK = 39
STG = 1360
CB = STG + C
OWN = N // NSC
HALF = 25088
WB = HALF // NTILE
ZR = 56


def _sc_scatter(edge_index, edge_weight, x):
    mesh = plsc.VectorSubcoreMesh(core_axis_name="c", subcore_axis_name="s",
                                  num_cores=NSC, num_subcores=NTILE)

    @functools.partial(
        pl.kernel,
        out_type=jax.ShapeDtypeStruct((NSC, HALF, D), jnp.float32),
        mesh=mesh,
        scratch_types=[
            pltpu.VMEM((STG,), jnp.int32),
            pltpu.VMEM((STG,), jnp.int32),
            pltpu.VMEM((STG,), jnp.float32),
            pltpu.VMEM((CB,), jnp.int32),
            pltpu.VMEM((CB,), jnp.int32),
            pltpu.VMEM((CB,), jnp.float32),
            pltpu.VMEM((C, D), jnp.float32),
            pltpu.VMEM((C, D), jnp.float32),
            pltpu.VMEM((C,), jnp.int32),
            pltpu.VMEM((C,), jnp.int32),
            pltpu.VMEM((ZR, D), jnp.float32),
            pltpu.VMEM_SHARED((HALF, D), jnp.float32),
            pltpu.SemaphoreType.DMA,
            pltpu.SemaphoreType.DMA,
            pltpu.SemaphoreType.DMA,
        ],
        compiler_params=pltpu.CompilerParams(use_tc_tiling_on_sc=False,
                                             needs_layout_passes=False),
    )
    def k(ei_hbm, w_hbm, x_hbm, out_hbm,
          src_f, dst_f, w_f, src_c, dst_c, w_c, rows_a, rows_b,
          sidx_a, sidx_b, zbuf, accum, isem, gsem_a, gsem_b):
        cc = lax.axis_index("c")
        s = lax.axis_index("s")
        base = cc * OWN

        def zfill(r, carry):
            for g in range(D // 16):
                zbuf[r, pl.ds(g * 16, 16)] = jnp.zeros((16,), jnp.float32)
            return carry
        lax.fori_loop(0, ZR, zfill, 0)
        for z in range(WB // ZR):
            pltpu.sync_copy(zbuf, accum.at[pl.ds(s * WB + z * ZR, ZR)])
        plsc.subcore_barrier()

        def issue(j, rows_buf, gsem):
            pltpu.async_copy(x_hbm.at[src_c.at[pl.ds(j * C, C)]],
                             rows_buf, gsem)

        def process(j, rows_buf, sidx_buf, gsem):
            for v in range(C // 16):
                sl = pl.ds(v * 16, 16)
                sidx_buf[sl] = dst_c[pl.ds(j * C + v * 16, 16)]
            pltpu.make_async_copy(x_hbm.at[src_c.at[pl.ds(j * C, C)]],
                                  rows_buf, gsem).wait()
            for v16 in range(C // 16):
                wv = w_c[pl.ds(j * C + v16 * 16, 16)]
                for l in range(16):
                    e = v16 * 16 + l
                    ws = wv[l]
                    for v in range(D // 16):
                        sl = pl.ds(v * 16, 16)
                        rows_buf[e, sl] = rows_buf[e, sl] * ws
            pltpu.sync_copy(rows_buf, accum.at[sidx_buf], add=True)

        def block_body(b, carry):
            e0 = s * TE + b * BLK
            d1 = pltpu.async_copy(ei_hbm.at[0, pl.ds(e0, STG)], src_f, isem)
            d2 = pltpu.async_copy(ei_hbm.at[1, pl.ds(e0, STG)], dst_f, isem)
            d3 = pltpu.async_copy(w_hbm.at[pl.ds(e0, STG)], w_f, isem)
            d1.wait()
            d2.wait()
            d3.wait()

            ngroups = jnp.where(b == NBLK - 1, STG // 16, BLK // 16)

            def comp_group(g, off):
                sl = pl.ds(g * 16, 16)
                loc = dst_f[sl] - base
                ok = (loc >= 0) & (loc < OWN)
                plsc.store_compressed(src_c.at[pl.ds(off, 16)],
                                      src_f[sl], mask=ok)
                plsc.store_compressed(dst_c.at[pl.ds(off, 16)],
                                      loc, mask=ok)
                plsc.store_compressed(w_c.at[pl.ds(off, 16)],
                                      w_f[sl], mask=ok)
                return off + plsc.all_reduce_population_count(ok)[0]
            cnt = lax.fori_loop(0, ngroups, comp_group, jnp.int32(0))

            zi = jnp.zeros((16,), jnp.int32)
            zf = jnp.zeros((16,), jnp.float32)
            full = zi == zi
            for g in range(C // 16):
                plsc.store_compressed(src_c.at[pl.ds(cnt + g * 16, 16)],
                                      zi, mask=full)
                plsc.store_compressed(dst_c.at[pl.ds(cnt + g * 16, 16)],
                                      zi, mask=full)
                plsc.store_compressed(w_c.at[pl.ds(cnt + g * 16, 16)],
                                      zf, mask=full)

            nchunks = lax.div(cnt + (C - 1), jnp.int32(C))

            @pl.when(nchunks > 0)
            def _prime():
                issue(0, rows_a, gsem_a)

            def pair_body(p, inner):
                j0 = 2 * p
                issue(j0 + 1, rows_b, gsem_b)
                process(j0, rows_a, sidx_a, gsem_a)

                @pl.when(j0 + 2 < nchunks)
                def _prefetch_a():
                    issue(j0 + 2, rows_a, gsem_a)
                process(j0 + 1, rows_b, sidx_b, gsem_b)
                return inner
            lax.fori_loop(0, nchunks // 2, pair_body, 0)

            @pl.when(lax.rem(nchunks, jnp.int32(2)) == 1)
            def _last_odd():
                process(nchunks - 1, rows_a, sidx_a, gsem_a)
            return carry
        lax.fori_loop(0, NBLK, block_body, 0)

        plsc.subcore_barrier()
        pltpu.sync_copy(accum.at[pl.ds(s * WB, WB)],
                        out_hbm.at[cc].at[pl.ds(s * WB, WB)])

    return k(edge_index, edge_weight, x)


BM = 1000
BPH = OWN // BM


def _dense_block(sd, xx, w1, b1, w2, b2):
    y = jnp.dot(sd + xx, w1, preferred_element_type=jnp.float32)
    y = y + jnp.dot(sd * xx, w2, preferred_element_type=jnp.float32)
    y = y + b1 + b2
    y = jnp.where(y >= 0, y, 0.01 * y)
    nrm = jnp.sqrt(jnp.sum(y * y, axis=1, keepdims=True))
    return y / jnp.maximum(nrm, 1e-12)


def _tc_dense1(acc, x, W1, b1, W2, b2):
    def body(side_ref, x_ref, w1_ref, b1_ref, w2_ref, b2_ref,
             o192_ref, x1_ref):
        xx = x_ref[...]
        y = _dense_block(side_ref[0], xx, w1_ref[...], b1_ref[...],
                         w2_ref[...], b2_ref[...])
        o192_ref[...] = jnp.concatenate([xx, y, y], axis=1)
        x1_ref[...] = y

    return pl.pallas_call(
        body,
        grid=(N // BM,),
        in_specs=[
            pl.BlockSpec((1, BM, D), lambda i: (i // BPH, i % BPH, 0)),
            pl.BlockSpec((BM, D), lambda i: (i, 0)),
            pl.BlockSpec((D, D), lambda i: (0, 0)),
            pl.BlockSpec((1, D), lambda i: (0, 0)),
            pl.BlockSpec((D, D), lambda i: (0, 0)),
            pl.BlockSpec((1, D), lambda i: (0, 0)),
        ],
        out_specs=(pl.BlockSpec((BM, 3 * D), lambda i: (i, 0)),
                   pl.BlockSpec((BM, D), lambda i: (i, 0))),
        out_shape=(jax.ShapeDtypeStruct((N, 3 * D), jnp.float32),
                   jax.ShapeDtypeStruct((N, D), jnp.float32)),
    )(acc, x, W1, b1.reshape(1, D), W2, b2.reshape(1, D))


def _tc_dense2(acc, x1, prev192, W1, b1, W2, b2):
    def body(side_ref, x_ref, p_ref, w1_ref, b1_ref, w2_ref, b2_ref, o_ref):
        y = _dense_block(side_ref[0], x_ref[...], w1_ref[...], b1_ref[...],
                         w2_ref[...], b2_ref[...])
        o_ref[...] = jnp.concatenate([p_ref[:, 0:2 * D], y], axis=1)

    return pl.pallas_call(
        body,
        grid=(N // BM,),
        in_specs=[
            pl.BlockSpec((1, BM, D), lambda i: (i // BPH, i % BPH, 0)),
            pl.BlockSpec((BM, D), lambda i: (i, 0)),
            pl.BlockSpec((BM, 3 * D), lambda i: (i, 0)),
            pl.BlockSpec((D, D), lambda i: (0, 0)),
            pl.BlockSpec((1, D), lambda i: (0, 0)),
            pl.BlockSpec((D, D), lambda i: (0, 0)),
            pl.BlockSpec((1, D), lambda i: (0, 0)),
        ],
        out_specs=pl.BlockSpec((BM, 3 * D), lambda i: (i, 0)),
        out_shape=jax.ShapeDtypeStruct((N, 3 * D), jnp.float32),
    )(acc, x1, prev192, W1, b1.reshape(1, D), W2, b2.reshape(1, D))


def kernel(edge_index, edge_weight, emb,
           W1_0, b1_0, W2_0, b2_0, W1_1, b1_1, W2_1, b2_1):
    acc1 = _sc_scatter(edge_index, edge_weight, emb)
    out192, x1 = _tc_dense1(acc1, emb, W1_0, b1_0, W2_0, b2_0)
    acc2 = _sc_scatter(edge_index, edge_weight, x1)
    return _tc_dense2(acc2, x1, out192, W1_1, b1_1, W2_1, b2_1)


# speedup vs baseline: 4.5786x; 1.3686x over previous
import functools

import jax
import jax.numpy as jnp
from jax import lax
from jax.experimental import pallas as pl
from jax.experimental.pallas import tpu as pltpu
from jax.experimental.pallas import tpu_sc as plsc

N = 50000
E = 800000
D = 64

C = 80
NSC = 2
NTILE = 16
TE = E // (NTILE)
BLK = 2000
NBLK = TE // BLK
CB = BLK + C
OWN = N // NSC
HALF = 25088
WB = HALF // NTILE
ZR = 56


def _sc_scatter(edge_index, edge_weight, x):
    mesh = plsc.VectorSubcoreMesh(core_axis_name="c", subcore_axis_name="s",
                                  num_cores=NSC, num_subcores=NTILE)

    @functools.partial(
        pl.kernel,
        out_type=jax.ShapeDtypeStruct((NSC, HALF, D), jnp.float32),
        mesh=mesh,
        scratch_types=[
            pltpu.VMEM((BLK,), jnp.int32),
            pltpu.VMEM((BLK,), jnp.int32),
            pltpu.VMEM((BLK,), jnp.float32),
            pltpu.VMEM((CB,), jnp.int32),
            pltpu.VMEM((CB,), jnp.int32),
            pltpu.VMEM((CB,), jnp.float32),
            pltpu.VMEM((C, D), jnp.float32),
            pltpu.VMEM((C, D), jnp.float32),
            pltpu.VMEM((C,), jnp.int32),
            pltpu.VMEM((C,), jnp.int32),
            pltpu.VMEM((ZR, D), jnp.float32),
            pltpu.VMEM_SHARED((HALF, D), jnp.float32),
            pltpu.SemaphoreType.DMA,
            pltpu.SemaphoreType.DMA,
            pltpu.SemaphoreType.DMA,
        ],
        compiler_params=pltpu.CompilerParams(use_tc_tiling_on_sc=False,
                                             needs_layout_passes=False),
    )
    def k(ei_hbm, w_hbm, x_hbm, out_hbm,
          src_f, dst_f, w_f, src_c, dst_c, w_c, rows_a, rows_b,
          sidx_a, sidx_b, zbuf, accum, isem, gsem_a, gsem_b):
        cc = lax.axis_index("c")
        s = lax.axis_index("s")
        base = cc * OWN

        def zfill(r, carry):
            for g in range(D // 16):
                zbuf[r, pl.ds(g * 16, 16)] = jnp.zeros((16,), jnp.float32)
            return carry
        lax.fori_loop(0, ZR, zfill, 0)
        for z in range(WB // ZR):
            pltpu.sync_copy(zbuf, accum.at[pl.ds(s * WB + z * ZR, ZR)])
        plsc.subcore_barrier()

        def issue(j, rows_buf, gsem):
            pltpu.async_copy(x_hbm.at[src_c.at[pl.ds(j * C, C)]],
                             rows_buf, gsem)

        def process(j, rows_buf, sidx_buf, gsem):
            for v in range(C // 16):
                sl = pl.ds(v * 16, 16)
                sidx_buf[sl] = dst_c[pl.ds(j * C + v * 16, 16)]
            pltpu.make_async_copy(x_hbm.at[src_c.at[pl.ds(j * C, C)]],
                                  rows_buf, gsem).wait()
            for v16 in range(C // 16):
                wv = w_c[pl.ds(j * C + v16 * 16, 16)]
                for l in range(16):
                    e = v16 * 16 + l
                    ws = wv[l]
                    for v in range(D // 16):
                        sl = pl.ds(v * 16, 16)
                        rows_buf[e, sl] = rows_buf[e, sl] * ws
            pltpu.sync_copy(rows_buf, accum.at[sidx_buf], add=True)

        def block_body(b, carry):
            e0 = s * TE + b * BLK
            d1 = pltpu.async_copy(ei_hbm.at[0, pl.ds(e0, BLK)], src_f, isem)
            d2 = pltpu.async_copy(ei_hbm.at[1, pl.ds(e0, BLK)], dst_f, isem)
            d3 = pltpu.async_copy(w_hbm.at[pl.ds(e0, BLK)], w_f, isem)
            d1.wait()
            d2.wait()
            d3.wait()

            def comp_row(r, off):
                for gg in range(5):
                    sl = pl.ds((r * 5 + gg) * 16, 16)
                    loc = dst_f[sl] - base
                    ok = (loc >= 0) & (loc < OWN)
                    plsc.store_compressed(src_c.at[pl.ds(off, 16)],
                                          src_f[sl], mask=ok)
                    plsc.store_compressed(dst_c.at[pl.ds(off, 16)],
                                          loc, mask=ok)
                    plsc.store_compressed(w_c.at[pl.ds(off, 16)],
                                          w_f[sl], mask=ok)
                    off = off + plsc.all_reduce_population_count(ok)[0]
                return off
            cnt = lax.fori_loop(0, BLK // 80, comp_row, jnp.int32(0))

            zi = jnp.zeros((16,), jnp.int32)
            zf = jnp.zeros((16,), jnp.float32)
            full = zi == zi
            for g in range(C // 16):
                plsc.store_compressed(src_c.at[pl.ds(cnt + g * 16, 16)],
                                      zi, mask=full)
                plsc.store_compressed(dst_c.at[pl.ds(cnt + g * 16, 16)],
                                      zi, mask=full)
                plsc.store_compressed(w_c.at[pl.ds(cnt + g * 16, 16)],
                                      zf, mask=full)

            nchunks = lax.div(cnt + (C - 1), jnp.int32(C))

            @pl.when(nchunks > 0)
            def _prime():
                issue(0, rows_a, gsem_a)

            def pair_body(p, inner):
                j0 = 2 * p
                issue(j0 + 1, rows_b, gsem_b)
                process(j0, rows_a, sidx_a, gsem_a)

                @pl.when(j0 + 2 < nchunks)
                def _prefetch_a():
                    issue(j0 + 2, rows_a, gsem_a)
                process(j0 + 1, rows_b, sidx_b, gsem_b)
                return inner
            lax.fori_loop(0, nchunks // 2, pair_body, 0)

            @pl.when(lax.rem(nchunks, jnp.int32(2)) == 1)
            def _last_odd():
                process(nchunks - 1, rows_a, sidx_a, gsem_a)
            return carry
        lax.fori_loop(0, NBLK, block_body, 0)

        plsc.subcore_barrier()
        pltpu.sync_copy(accum.at[pl.ds(s * WB, WB)],
                        out_hbm.at[cc].at[pl.ds(s * WB, WB)])

    return k(edge_index, edge_weight, x)


BM = 1000
BPH = OWN // BM


def _dense_block(sd, xx, w1, b1, w2, b2):
    y = jnp.dot(sd + xx, w1, preferred_element_type=jnp.float32)
    y = y + jnp.dot(sd * xx, w2, preferred_element_type=jnp.float32)
    y = y + b1 + b2
    y = jnp.where(y >= 0, y, 0.01 * y)
    nrm = jnp.sqrt(jnp.sum(y * y, axis=1, keepdims=True))
    return y / jnp.maximum(nrm, 1e-12)


def _tc_dense1(acc, x, W1, b1, W2, b2):
    def body(side_ref, x_ref, w1_ref, b1_ref, w2_ref, b2_ref,
             o192_ref, x1_ref):
        xx = x_ref[...]
        y = _dense_block(side_ref[0], xx, w1_ref[...], b1_ref[...],
                         w2_ref[...], b2_ref[...])
        o192_ref[...] = jnp.concatenate([xx, y, y], axis=1)
        x1_ref[...] = y

    return pl.pallas_call(
        body,
        grid=(N // BM,),
        in_specs=[
            pl.BlockSpec((1, BM, D), lambda i: (i // BPH, i % BPH, 0)),
            pl.BlockSpec((BM, D), lambda i: (i, 0)),
            pl.BlockSpec((D, D), lambda i: (0, 0)),
            pl.BlockSpec((1, D), lambda i: (0, 0)),
            pl.BlockSpec((D, D), lambda i: (0, 0)),
            pl.BlockSpec((1, D), lambda i: (0, 0)),
        ],
        out_specs=(pl.BlockSpec((BM, 3 * D), lambda i: (i, 0)),
                   pl.BlockSpec((BM, D), lambda i: (i, 0))),
        out_shape=(jax.ShapeDtypeStruct((N, 3 * D), jnp.float32),
                   jax.ShapeDtypeStruct((N, D), jnp.float32)),
    )(acc, x, W1, b1.reshape(1, D), W2, b2.reshape(1, D))


def _tc_dense2(acc, x1, prev192, W1, b1, W2, b2):
    def body(side_ref, x_ref, p_ref, w1_ref, b1_ref, w2_ref, b2_ref, o_ref):
        y = _dense_block(side_ref[0], x_ref[...], w1_ref[...], b1_ref[...],
                         w2_ref[...], b2_ref[...])
        o_ref[...] = jnp.concatenate([p_ref[:, 0:2 * D], y], axis=1)

    return pl.pallas_call(
        body,
        grid=(N // BM,),
        in_specs=[
            pl.BlockSpec((1, BM, D), lambda i: (i // BPH, i % BPH, 0)),
            pl.BlockSpec((BM, D), lambda i: (i, 0)),
            pl.BlockSpec((BM, 3 * D), lambda i: (i, 0)),
            pl.BlockSpec((D, D), lambda i: (0, 0)),
            pl.BlockSpec((1, D), lambda i: (0, 0)),
            pl.BlockSpec((D, D), lambda i: (0, 0)),
            pl.BlockSpec((1, D), lambda i: (0, 0)),
        ],
        out_specs=pl.BlockSpec((BM, 3 * D), lambda i: (i, 0)),
        out_shape=jax.ShapeDtypeStruct((N, 3 * D), jnp.float32),
    )(acc, x1, prev192, W1, b1.reshape(1, D), W2, b2.reshape(1, D))


def kernel(edge_index, edge_weight, emb,
           W1_0, b1_0, W2_0, b2_0, W1_1, b1_1, W2_1, b2_1):
    acc1 = _sc_scatter(edge_index, edge_weight, emb)
    out192, x1 = _tc_dense1(acc1, emb, W1_0, b1_0, W2_0, b2_0)
    acc2 = _sc_scatter(edge_index, edge_weight, x1)
    return _tc_dense2(acc2, x1, out192, W1_1, b1_1, W2_1, b2_1)


# speedup vs baseline: 5.2403x; 1.1445x over previous
import functools

import jax
import jax.numpy as jnp
from jax import lax
from jax.experimental import pallas as pl
from jax.experimental.pallas import tpu as pltpu
from jax.experimental.pallas import tpu_sc as plsc

N = 50000
E = 800000
D = 64

C = 80
EP = 819200
ROWSP = EP // C
NSC = 2
NTILE = 16
TROWS = ROWSP // NTILE
NB = 32
NBLK = TROWS // NB
CB = NB * C + C
OWN = N // NSC
HALF = 25088
WB = HALF // NTILE
ZR = 56


def _sc_scatter(src_r, dst_r, w_r, x):
    mesh = plsc.VectorSubcoreMesh(core_axis_name="c", subcore_axis_name="s",
                                  num_cores=NSC, num_subcores=NTILE)

    @functools.partial(
        pl.kernel,
        out_type=jax.ShapeDtypeStruct((NSC, HALF, D), jnp.float32),
        mesh=mesh,
        scratch_types=[
            pltpu.VMEM((NB, C), jnp.int32),
            pltpu.VMEM((NB, C), jnp.int32),
            pltpu.VMEM((NB, C), jnp.float32),
            pltpu.VMEM((CB,), jnp.int32),
            pltpu.VMEM((CB,), jnp.int32),
            pltpu.VMEM((CB,), jnp.float32),
            pltpu.VMEM((C, D), jnp.float32),
            pltpu.VMEM((C, D), jnp.float32),
            pltpu.VMEM((C,), jnp.int32),
            pltpu.VMEM((C,), jnp.int32),
            pltpu.VMEM((ZR, D), jnp.float32),
            pltpu.VMEM_SHARED((HALF, D), jnp.float32),
            pltpu.SemaphoreType.DMA,
            pltpu.SemaphoreType.DMA,
            pltpu.SemaphoreType.DMA,
        ],
        compiler_params=pltpu.CompilerParams(use_tc_tiling_on_sc=False,
                                             needs_layout_passes=False),
    )
    def k(src_hbm, dst_hbm, w_hbm, x_hbm, out_hbm,
          src_v, dst_v, w_v, src_c, dst_c, w_c, rows_a, rows_b,
          sidx_a, sidx_b, zbuf, accum, isem, gsem_a, gsem_b):
        cc = lax.axis_index("c")
        s = lax.axis_index("s")
        base = cc * OWN

        def zfill(r, carry):
            for g in range(D // 16):
                zbuf[r, pl.ds(g * 16, 16)] = jnp.zeros((16,), jnp.float32)
            return carry
        lax.fori_loop(0, ZR, zfill, 0)
        for z in range(WB // ZR):
            pltpu.sync_copy(zbuf, accum.at[pl.ds(s * WB + z * ZR, ZR)])
        plsc.subcore_barrier()

        def issue(j, rows_buf, gsem):
            pltpu.async_copy(x_hbm.at[src_c.at[pl.ds(j * C, C)]],
                             rows_buf, gsem)

        def process(j, rows_buf, sidx_buf, gsem):
            for v in range(C // 16):
                sl = pl.ds(v * 16, 16)
                sidx_buf[sl] = dst_c[pl.ds(j * C + v * 16, 16)]
            pltpu.make_async_copy(x_hbm.at[src_c.at[pl.ds(j * C, C)]],
                                  rows_buf, gsem).wait()
            for v16 in range(C // 16):
                wv = w_c[pl.ds(j * C + v16 * 16, 16)]
                for l in range(16):
                    e = v16 * 16 + l
                    ws = wv[l]
                    for v in range(D // 16):
                        sl = pl.ds(v * 16, 16)
                        rows_buf[e, sl] = rows_buf[e, sl] * ws
            pltpu.sync_copy(rows_buf, accum.at[sidx_buf], add=True)

        def block_body(b, carry):
            r0 = s * TROWS + b * NB
            d1 = pltpu.async_copy(src_hbm.at[pl.ds(r0, NB)], src_v, isem)
            d2 = pltpu.async_copy(dst_hbm.at[pl.ds(r0, NB)], dst_v, isem)
            d3 = pltpu.async_copy(w_hbm.at[pl.ds(r0, NB)], w_v, isem)
            d1.wait()
            d2.wait()
            d3.wait()

            def comp_row(r, off):
                for gg in range(C // 16):
                    sl = pl.ds(gg * 16, 16)
                    loc = dst_v[r, sl] - base
                    ok = (loc >= 0) & (loc < OWN)
                    plsc.store_compressed(src_c.at[pl.ds(off, 16)],
                                          src_v[r, sl], mask=ok)
                    plsc.store_compressed(dst_c.at[pl.ds(off, 16)],
                                          loc, mask=ok)
                    plsc.store_compressed(w_c.at[pl.ds(off, 16)],
                                          w_v[r, sl], mask=ok)
                    off = off + plsc.all_reduce_population_count(ok)[0]
                return off
            cnt = lax.fori_loop(0, NB, comp_row, jnp.int32(0))

            zi = jnp.zeros((16,), jnp.int32)
            zf = jnp.zeros((16,), jnp.float32)
            full = zi == zi
            for g in range(C // 16):
                plsc.store_compressed(src_c.at[pl.ds(cnt + g * 16, 16)],
                                      zi, mask=full)
                plsc.store_compressed(dst_c.at[pl.ds(cnt + g * 16, 16)],
                                      zi, mask=full)
                plsc.store_compressed(w_c.at[pl.ds(cnt + g * 16, 16)],
                                      zf, mask=full)

            nchunks = lax.div(cnt + (C - 1), jnp.int32(C))

            @pl.when(nchunks > 0)
            def _prime():
                issue(0, rows_a, gsem_a)

            def pair_body(p, inner):
                j0 = 2 * p
                issue(j0 + 1, rows_b, gsem_b)
                process(j0, rows_a, sidx_a, gsem_a)

                @pl.when(j0 + 2 < nchunks)
                def _prefetch_a():
                    issue(j0 + 2, rows_a, gsem_a)
                process(j0 + 1, rows_b, sidx_b, gsem_b)
                return inner
            lax.fori_loop(0, nchunks // 2, pair_body, 0)

            @pl.when(lax.rem(nchunks, jnp.int32(2)) == 1)
            def _last_odd():
                process(nchunks - 1, rows_a, sidx_a, gsem_a)
            return carry
        lax.fori_loop(0, NBLK, block_body, 0)

        plsc.subcore_barrier()
        pltpu.sync_copy(accum.at[pl.ds(s * WB, WB)],
                        out_hbm.at[cc].at[pl.ds(s * WB, WB)])

    return k(src_r, dst_r, w_r, x)


BM = 1000
BPH = OWN // BM


def _dense_block(sd, xx, w1, b1, w2, b2):
    y = jnp.dot(sd + xx, w1, preferred_element_type=jnp.float32)
    y = y + jnp.dot(sd * xx, w2, preferred_element_type=jnp.float32)
    y = y + b1 + b2
    y = jnp.where(y >= 0, y, 0.01 * y)
    nrm = jnp.sqrt(jnp.sum(y * y, axis=1, keepdims=True))
    return y / jnp.maximum(nrm, 1e-12)


def _tc_dense1(acc, x, W1, b1, W2, b2):
    def body(side_ref, x_ref, w1_ref, b1_ref, w2_ref, b2_ref,
             o192_ref, x1_ref):
        xx = x_ref[...]
        y = _dense_block(side_ref[0], xx, w1_ref[...], b1_ref[...],
                         w2_ref[...], b2_ref[...])
        o192_ref[...] = jnp.concatenate([xx, y, y], axis=1)
        x1_ref[...] = y

    return pl.pallas_call(
        body,
        grid=(N // BM,),
        in_specs=[
            pl.BlockSpec((1, BM, D), lambda i: (i // BPH, i % BPH, 0)),
            pl.BlockSpec((BM, D), lambda i: (i, 0)),
            pl.BlockSpec((D, D), lambda i: (0, 0)),
            pl.BlockSpec((1, D), lambda i: (0, 0)),
            pl.BlockSpec((D, D), lambda i: (0, 0)),
            pl.BlockSpec((1, D), lambda i: (0, 0)),
        ],
        out_specs=(pl.BlockSpec((BM, 3 * D), lambda i: (i, 0)),
                   pl.BlockSpec((BM, D), lambda i: (i, 0))),
        out_shape=(jax.ShapeDtypeStruct((N, 3 * D), jnp.float32),
                   jax.ShapeDtypeStruct((N, D), jnp.float32)),
    )(acc, x, W1, b1.reshape(1, D), W2, b2.reshape(1, D))


def _tc_dense2(acc, x1, prev192, W1, b1, W2, b2):
    def body(side_ref, x_ref, p_ref, w1_ref, b1_ref, w2_ref, b2_ref, o_ref):
        y = _dense_block(side_ref[0], x_ref[...], w1_ref[...], b1_ref[...],
                         w2_ref[...], b2_ref[...])
        o_ref[...] = jnp.concatenate([p_ref[:, 0:2 * D], y], axis=1)

    return pl.pallas_call(
        body,
        grid=(N // BM,),
        in_specs=[
            pl.BlockSpec((1, BM, D), lambda i: (i // BPH, i % BPH, 0)),
            pl.BlockSpec((BM, D), lambda i: (i, 0)),
            pl.BlockSpec((BM, 3 * D), lambda i: (i, 0)),
            pl.BlockSpec((D, D), lambda i: (0, 0)),
            pl.BlockSpec((1, D), lambda i: (0, 0)),
            pl.BlockSpec((D, D), lambda i: (0, 0)),
            pl.BlockSpec((1, D), lambda i: (0, 0)),
        ],
        out_specs=pl.BlockSpec((BM, 3 * D), lambda i: (i, 0)),
        out_shape=jax.ShapeDtypeStruct((N, 3 * D), jnp.float32),
    )(acc, x1, prev192, W1, b1.reshape(1, D), W2, b2.reshape(1, D))


def kernel(edge_index, edge_weight, emb,
           W1_0, b1_0, W2_0, b2_0, W1_1, b1_1, W2_1, b2_1):
    pad = EP - E
    src_r = jnp.concatenate(
        [edge_index[0], jnp.zeros((pad,), jnp.int32)]).reshape(ROWSP, C)
    dst_r = jnp.concatenate(
        [edge_index[1],
         jnp.full((pad,), N, jnp.int32)]).reshape(ROWSP, C)
    w_r = jnp.concatenate(
        [edge_weight, jnp.zeros((pad,), jnp.float32)]).reshape(ROWSP, C)
    acc1 = _sc_scatter(src_r, dst_r, w_r, emb)
    out192, x1 = _tc_dense1(acc1, emb, W1_0, b1_0, W2_0, b2_0)
    acc2 = _sc_scatter(src_r, dst_r, w_r, x1)
    return _tc_dense2(acc2, x1, out192, W1_1, b1_1, W2_1, b2_1)


# speedup vs baseline: 5.2579x; 1.0034x over previous
import functools

import jax
import jax.numpy as jnp
from jax import lax
from jax.experimental import pallas as pl
from jax.experimental.pallas import tpu as pltpu
from jax.experimental.pallas import tpu_sc as plsc

N = 50000
E = 800000
D = 64

C = 80
EP = 819200
ROWSP = EP // C
NSC = 2
NTILE = 16
TROWS = ROWSP // NTILE
NB = 32
NBLK = TROWS // NB
CB = NB * C + C
OWN = N // NSC
HALF = 25088
WB = HALF // NTILE
ZR = 56


def _sc_scatter(src_r, dst_r, w_r, x):
    mesh = plsc.VectorSubcoreMesh(core_axis_name="c", subcore_axis_name="s",
                                  num_cores=NSC, num_subcores=NTILE)

    @functools.partial(
        pl.kernel,
        out_type=jax.ShapeDtypeStruct((NSC, HALF, D), jnp.float32),
        mesh=mesh,
        scratch_types=[
            pltpu.VMEM((NB, C), jnp.int32),
            pltpu.VMEM((NB, C), jnp.int32),
            pltpu.VMEM((NB, C), jnp.float32),
            pltpu.VMEM((CB,), jnp.int32),
            pltpu.VMEM((CB,), jnp.int32),
            pltpu.VMEM((CB,), jnp.float32),
            pltpu.VMEM((C, D), jnp.float32),
            pltpu.VMEM((C, D), jnp.float32),
            pltpu.VMEM((C,), jnp.int32),
            pltpu.VMEM((C,), jnp.int32),
            pltpu.VMEM((ZR, D), jnp.float32),
            pltpu.VMEM_SHARED((HALF, D), jnp.float32),
            pltpu.SemaphoreType.DMA,
            pltpu.SemaphoreType.DMA,
            pltpu.SemaphoreType.DMA,
        ],
        compiler_params=pltpu.CompilerParams(use_tc_tiling_on_sc=False,
                                             needs_layout_passes=False),
    )
    def k(src_hbm, dst_hbm, w_hbm, x_hbm, out_hbm,
          src_v, dst_v, w_v, src_c, dst_c, w_c, rows_a, rows_b,
          sidx_a, sidx_b, zbuf, accum, isem, gsem_a, gsem_b):
        cc = lax.axis_index("c")
        s = lax.axis_index("s")
        base = cc * OWN

        def zfill(r, carry):
            for g in range(D // 16):
                zbuf[r, pl.ds(g * 16, 16)] = jnp.zeros((16,), jnp.float32)
            return carry
        lax.fori_loop(0, ZR, zfill, 0)
        for z in range(WB // ZR):
            pltpu.sync_copy(zbuf, accum.at[pl.ds(s * WB + z * ZR, ZR)])
        plsc.subcore_barrier()

        def issue(j, rows_buf, gsem):
            pltpu.async_copy(x_hbm.at[src_c.at[pl.ds(j * C, C)]],
                             rows_buf, gsem)

        def process(j, rows_buf, sidx_buf, gsem):
            for v in range(C // 16):
                sl = pl.ds(v * 16, 16)
                sidx_buf[sl] = dst_c[pl.ds(j * C + v * 16, 16)]
            pltpu.make_async_copy(x_hbm.at[src_c.at[pl.ds(j * C, C)]],
                                  rows_buf, gsem).wait()
            for v16 in range(C // 16):
                wv = w_c[pl.ds(j * C + v16 * 16, 16)]
                for l in range(16):
                    e = v16 * 16 + l
                    wsv = wv.at[jnp.full((16,), l, jnp.int32)].get(
                        mode="promise_in_bounds")
                    for v in range(D // 16):
                        sl = pl.ds(v * 16, 16)
                        rows_buf[e, sl] = rows_buf[e, sl] * wsv
            pltpu.sync_copy(rows_buf, accum.at[sidx_buf], add=True)

        def block_body(b, carry):
            r0 = s * TROWS + b * NB
            d1 = pltpu.async_copy(src_hbm.at[pl.ds(r0, NB)], src_v, isem)
            d2 = pltpu.async_copy(dst_hbm.at[pl.ds(r0, NB)], dst_v, isem)
            d3 = pltpu.async_copy(w_hbm.at[pl.ds(r0, NB)], w_v, isem)
            d1.wait()
            d2.wait()
            d3.wait()

            def comp_row(r, off):
                for gg in range(C // 16):
                    sl = pl.ds(gg * 16, 16)
                    loc = dst_v[r, sl] - base
                    ok = (loc >= 0) & (loc < OWN)
                    plsc.store_compressed(src_c.at[pl.ds(off, 16)],
                                          src_v[r, sl], mask=ok)
                    plsc.store_compressed(dst_c.at[pl.ds(off, 16)],
                                          loc, mask=ok)
                    plsc.store_compressed(w_c.at[pl.ds(off, 16)],
                                          w_v[r, sl], mask=ok)
                    off = off + plsc.all_reduce_population_count(ok)[0]
                return off
            cnt = lax.fori_loop(0, NB, comp_row, jnp.int32(0))

            zi = jnp.zeros((16,), jnp.int32)
            zf = jnp.zeros((16,), jnp.float32)
            full = zi == zi
            for g in range(C // 16):
                plsc.store_compressed(src_c.at[pl.ds(cnt + g * 16, 16)],
                                      zi, mask=full)
                plsc.store_compressed(dst_c.at[pl.ds(cnt + g * 16, 16)],
                                      zi, mask=full)
                plsc.store_compressed(w_c.at[pl.ds(cnt + g * 16, 16)],
                                      zf, mask=full)

            nchunks = lax.div(cnt + (C - 1), jnp.int32(C))

            @pl.when(nchunks > 0)
            def _prime():
                issue(0, rows_a, gsem_a)

            def pair_body(p, inner):
                j0 = 2 * p
                issue(j0 + 1, rows_b, gsem_b)
                process(j0, rows_a, sidx_a, gsem_a)

                @pl.when(j0 + 2 < nchunks)
                def _prefetch_a():
                    issue(j0 + 2, rows_a, gsem_a)
                process(j0 + 1, rows_b, sidx_b, gsem_b)
                return inner
            lax.fori_loop(0, nchunks // 2, pair_body, 0)

            @pl.when(lax.rem(nchunks, jnp.int32(2)) == 1)
            def _last_odd():
                process(nchunks - 1, rows_a, sidx_a, gsem_a)
            return carry
        lax.fori_loop(0, NBLK, block_body, 0)

        plsc.subcore_barrier()
        pltpu.sync_copy(accum.at[pl.ds(s * WB, WB)],
                        out_hbm.at[cc].at[pl.ds(s * WB, WB)])

    return k(src_r, dst_r, w_r, x)


BM = 1000
BPH = OWN // BM


def _dense_block(sd, xx, w1, b1, w2, b2):
    y = jnp.dot(sd + xx, w1, preferred_element_type=jnp.float32)
    y = y + jnp.dot(sd * xx, w2, preferred_element_type=jnp.float32)
    y = y + b1 + b2
    y = jnp.where(y >= 0, y, 0.01 * y)
    nrm = jnp.sqrt(jnp.sum(y * y, axis=1, keepdims=True))
    return y / jnp.maximum(nrm, 1e-12)


def _tc_dense1(acc, x, W1, b1, W2, b2):
    def body(side_ref, x_ref, w1_ref, b1_ref, w2_ref, b2_ref,
             o192_ref, x1_ref):
        xx = x_ref[...]
        y = _dense_block(side_ref[0], xx, w1_ref[...], b1_ref[...],
                         w2_ref[...], b2_ref[...])
        o192_ref[...] = jnp.concatenate([xx, y, y], axis=1)
        x1_ref[...] = y

    return pl.pallas_call(
        body,
        grid=(N // BM,),
        in_specs=[
            pl.BlockSpec((1, BM, D), lambda i: (i // BPH, i % BPH, 0)),
            pl.BlockSpec((BM, D), lambda i: (i, 0)),
            pl.BlockSpec((D, D), lambda i: (0, 0)),
            pl.BlockSpec((1, D), lambda i: (0, 0)),
            pl.BlockSpec((D, D), lambda i: (0, 0)),
            pl.BlockSpec((1, D), lambda i: (0, 0)),
        ],
        out_specs=(pl.BlockSpec((BM, 3 * D), lambda i: (i, 0)),
                   pl.BlockSpec((BM, D), lambda i: (i, 0))),
        out_shape=(jax.ShapeDtypeStruct((N, 3 * D), jnp.float32),
                   jax.ShapeDtypeStruct((N, D), jnp.float32)),
    )(acc, x, W1, b1.reshape(1, D), W2, b2.reshape(1, D))


def _tc_dense2(acc, x1, prev192, W1, b1, W2, b2):
    def body(side_ref, x_ref, p_ref, w1_ref, b1_ref, w2_ref, b2_ref, o_ref):
        y = _dense_block(side_ref[0], x_ref[...], w1_ref[...], b1_ref[...],
                         w2_ref[...], b2_ref[...])
        o_ref[...] = jnp.concatenate([p_ref[:, 0:2 * D], y], axis=1)

    return pl.pallas_call(
        body,
        grid=(N // BM,),
        in_specs=[
            pl.BlockSpec((1, BM, D), lambda i: (i // BPH, i % BPH, 0)),
            pl.BlockSpec((BM, D), lambda i: (i, 0)),
            pl.BlockSpec((BM, 3 * D), lambda i: (i, 0)),
            pl.BlockSpec((D, D), lambda i: (0, 0)),
            pl.BlockSpec((1, D), lambda i: (0, 0)),
            pl.BlockSpec((D, D), lambda i: (0, 0)),
            pl.BlockSpec((1, D), lambda i: (0, 0)),
        ],
        out_specs=pl.BlockSpec((BM, 3 * D), lambda i: (i, 0)),
        out_shape=jax.ShapeDtypeStruct((N, 3 * D), jnp.float32),
    )(acc, x1, prev192, W1, b1.reshape(1, D), W2, b2.reshape(1, D))


def kernel(edge_index, edge_weight, emb,
           W1_0, b1_0, W2_0, b2_0, W1_1, b1_1, W2_1, b2_1):
    pad = EP - E
    src_r = jnp.concatenate(
        [edge_index[0], jnp.zeros((pad,), jnp.int32)]).reshape(ROWSP, C)
    dst_r = jnp.concatenate(
        [edge_index[1],
         jnp.full((pad,), N, jnp.int32)]).reshape(ROWSP, C)
    w_r = jnp.concatenate(
        [edge_weight, jnp.zeros((pad,), jnp.float32)]).reshape(ROWSP, C)
    acc1 = _sc_scatter(src_r, dst_r, w_r, emb)
    out192, x1 = _tc_dense1(acc1, emb, W1_0, b1_0, W2_0, b2_0)
    acc2 = _sc_scatter(src_r, dst_r, w_r, x1)
    return _tc_dense2(acc2, x1, out192, W1_1, b1_1, W2_1, b2_1)
